# baseline (speedup 1.0000x reference)
import functools

import jax
import jax.numpy as jnp
from jax import lax
from jax.experimental import pallas as pl
from jax.experimental.pallas import tpu as pltpu
from jax.experimental.pallas import tpu_sc as plsc

_B = 4096
_F = 26
_V = 100000
_D = 32
_EO = _F * _D
_H = 64
_NF = _F * _V
_NC = 2
_NS = 16
_NW = _NC * _NS
_TOT = _B * _F
_CHUNK = _TOT // _NW
_STEP = 128
_NSTEP = _CHUNK // _STEP
_LANES = 16


def _sc_gather_body(emb_hbm, fc_hbm, x_hbm, offs_hbm,
                    emb_out, fc_out,
                    idx_v, offs_v, rows_v, fcv_v, sem_e, sem_f):
    wid = lax.axis_index("s") * _NC + lax.axis_index("c")
    base = pl.multiple_of(wid * _CHUNK, _CHUNK)

    pltpu.sync_copy(x_hbm.at[pl.ds(base, _CHUNK)], idx_v)
    pltpu.sync_copy(offs_hbm.at[:], offs_v)

    def _add_body(i, carry):
        sl = pl.ds(pl.multiple_of(i * _LANES, _LANES), _LANES)
        idx_v[sl] = idx_v[sl] + offs_v[sl]
        return carry
    lax.fori_loop(0, _CHUNK // _LANES, _add_body, 0)

    def _gather_body(j, carry):
        off = pl.ds(pl.multiple_of(j * _STEP, _STEP), _STEP)
        pltpu.async_copy(emb_hbm.at[idx_v.at[off]], rows_v.at[off], sem_e)
        pltpu.async_copy(fc_hbm.at[idx_v.at[off]], fcv_v.at[off], sem_f)
        return carry
    lax.fori_loop(0, _NSTEP, _gather_body, 0)

    pltpu.make_async_copy(emb_out.at[pl.ds(base, _CHUNK)], rows_v, sem_e).wait()
    pltpu.make_async_copy(fc_out.at[pl.ds(base, _CHUNK)], fcv_v, sem_f).wait()

    pltpu.sync_copy(rows_v, emb_out.at[pl.ds(base, _CHUNK)])
    pltpu.sync_copy(fcv_v, fc_out.at[pl.ds(base, _CHUNK)])


@functools.lru_cache(maxsize=1)
def _make_sc_gather():
    mesh = plsc.VectorSubcoreMesh(core_axis_name="c", subcore_axis_name="s")
    return pl.kernel(
        _sc_gather_body,
        out_type=[
            jax.ShapeDtypeStruct((_TOT, _D), jnp.float32),
            jax.ShapeDtypeStruct((_TOT,), jnp.float32),
        ],
        mesh=mesh,
        compiler_params=pltpu.CompilerParams(use_tc_tiling_on_sc=False),
        scratch_types=[
            pltpu.VMEM((_CHUNK,), jnp.int32),
            pltpu.VMEM((_CHUNK,), jnp.int32),
            pltpu.VMEM((_CHUNK, _D), jnp.float32),
            pltpu.VMEM((_CHUNK,), jnp.float32),
            pltpu.SemaphoreType.DMA,
            pltpu.SemaphoreType.DMA,
        ],
    )


_BB = 512
_NBB = _B // _BB


def _tc_body(emb_ref, fcg_ref, w1_ref, b1_ref, gamma_ref, beta_ref,
             w2_ref, b2_ref, linw_ref, linb_ref, out_ref, h_s, fl_s):
    ii = lax.broadcasted_iota(jnp.int32, (_EO, _D), 0)
    jj = lax.broadcasted_iota(jnp.int32, (_EO, _D), 1)
    S = jnp.where(ii % _D == jj, 1.0, 0.0).astype(jnp.float32)
    w1 = w1_ref[...]
    b1 = b1_ref[...]
    linw = linw_ref[0, 0]
    linb = linb_ref[0, 0]

    def _blk(b, carry):
        rows = pl.ds(b * _BB, _BB)
        e = emb_ref[rows, :]
        ssq = jnp.sum(e * e, axis=1, keepdims=True)
        s = lax.dot(e, S, precision=lax.Precision.HIGHEST)
        fm = 0.5 * (jnp.sum(s * s, axis=1, keepdims=True) - ssq)
        lin = (jnp.sum(fcg_ref[rows, :], axis=1, keepdims=True) * linw + linb)
        h_s[rows, :] = lax.dot(e, w1, precision=lax.Precision.HIGHEST) + b1
        fl_s[rows, :] = fm + lin
        return carry
    lax.fori_loop(0, _NBB, _blk, 0)

    h = h_s[...]
    mean = jnp.mean(h, axis=0, keepdims=True)
    var = jnp.mean(jnp.square(h - mean), axis=0, keepdims=True)
    hn = (h - mean) * lax.rsqrt(var + 1e-5) * gamma_ref[...] + beta_ref[...]
    hr = jnp.maximum(hn, 0.0)
    deep = lax.dot(hr, w2_ref[...], precision=lax.Precision.HIGHEST) + b2_ref[0, 0]

    out_ref[...] = fl_s[...] + deep


def _tc_forward(emb_flat, fcg, W1, b1, gamma, beta, W2, b2, lin_W, lin_b):
    return pl.pallas_call(
        _tc_body,
        out_shape=jax.ShapeDtypeStruct((_B, 1), jnp.float32),
        scratch_shapes=[
            pltpu.VMEM((_B, _H), jnp.float32),
            pltpu.VMEM((_B, 1), jnp.float32),
        ],
    )(emb_flat, fcg,
      W1, b1.reshape(1, _H), gamma.reshape(1, _H), beta.reshape(1, _H),
      W2, b2.reshape(1, 1), lin_W, lin_b.reshape(1, 1))


def kernel(emb_table, fc_table, lin_W, lin_b, W1, b1, gamma, beta, W2, b2, x):
    x_flat = x.reshape(-1)
    offs = jnp.tile(jnp.arange(_F, dtype=jnp.int32) * _V, _CHUNK // _F)
    emb_rows, fc_rows = _make_sc_gather()(
        emb_table, fc_table.reshape(-1), x_flat, offs)
    emb_flat = emb_rows.reshape(_B, _EO)
    fcg = fc_rows.reshape(_B, _F)
    out = _tc_forward(emb_flat, fcg, W1, b1, gamma, beta, W2, b2, lin_W, lin_b)
    return out.reshape(-1)


# speedup vs baseline: 2.6534x; 2.6534x over previous
import functools

import jax
import jax.numpy as jnp
from jax import lax
from jax.experimental import pallas as pl
from jax.experimental.pallas import tpu as pltpu
from jax.experimental.pallas import tpu_sc as plsc

_B = 4096
_F = 26
_V = 100000
_D = 32
_EO = _F * _D
_H = 64
_NF = _F * _V
_NC = 2
_NS = 16
_NW = _NC * _NS
_TOT = _B * _F
_WIN = 512
_NWIN = 197
_RING = 4
_SRING = 4
_SORT = 10240
_SROWS = _F * _SORT // 4
_PROWS = _TOT // 4
_STEP = 128
_LANES = 16
_BCH = 256
_NBCH = _TOT // _NW // _BCH


def _sc_stream_body(emb_t, fc_hbm, x_hbm, outs_hbm, inv_hbm, fc_out,
                    win_v, xv, idxg_v, fcv, sj_v, inv_v, stage_v,
                    cnt_smem, off_smem, fill_smem,
                    sem_s0, sem_s1, sem_s2, sem_s3,
                    sem_w0, sem_w1, sem_w2, sem_w3, sem_f):
    wid = lax.axis_index("s") * _NC + lax.axis_index("c")

    sem_s = (sem_s0, sem_s1, sem_s2, sem_s3)
    sem_w = (sem_w0, sem_w1, sem_w2, sem_w3)

    @pl.when(wid < _F)
    def _active():
        f = wid
        fV = f * _V
        r_f = lax.rem(fV, _WIN)
        base0 = fV - r_f
        pbase = pl.multiple_of(f * _B, _B)

        def _fire(k):
            raw = base0 + k * _WIN
            slot = lax.rem(k, _RING)
            fullc = raw + _WIN <= _NF
            for s in range(_RING):
                @pl.when(jnp.logical_and(slot == s, fullc))
                def _full(s=s):
                    pltpu.async_copy(
                        emb_t.at[:, pl.ds(pl.multiple_of(raw, _WIN), _WIN)],
                        win_v.at[s], sem_s[s])

                @pl.when(jnp.logical_and(slot == s, jnp.logical_not(fullc)))
                def _tail(s=s):
                    pltpu.async_copy(
                        emb_t.at[:, pl.ds(pl.multiple_of(raw, 128), 128)],
                        win_v.at[s, :, pl.ds(0, 128)], sem_s[s])

        pltpu.sync_copy(x_hbm.at[pl.ds(pbase, _B)], xv)
        for k0 in range(_RING):
            _fire(k0)

        def _zero(i, c):
            cnt_smem[i] = 0
            return c
        lax.fori_loop(0, 256, _zero, 0)

        def _cnt(i16, c):
            v = xv[pl.ds(pl.multiple_of(i16 * _LANES, _LANES), _LANES)]
            wv = lax.shift_right_logical(v + r_f, 9)
            for l in range(_LANES):
                w = wv[l]
                cnt_smem[w] = cnt_smem[w] + 1
            return c
        lax.fori_loop(0, _B // _LANES, _cnt, 0)

        def _offs(k, cur):
            off_smem[k] = cur
            fill_smem[k] = cur
            padded = lax.shift_right_logical(cnt_smem[k] + 31, 5) * 32
            return cur + padded
        lax.fori_loop(0, _NWIN, _offs, 0)

        lane0 = lax.iota(jnp.int32, _LANES) == 0

        def _scat(i16, c):
            v = xv[pl.ds(pl.multiple_of(i16 * _LANES, _LANES), _LANES)]
            wv16 = lax.shift_right_logical(v + r_f, 9)
            for l in range(_LANES):
                j = v[l]
                w = wv16[l]
                slot = fill_smem[w]
                fill_smem[w] = slot + 1
                plsc.store_scatter(
                    sj_v, [jnp.full((_LANES,), slot, dtype=jnp.int32)],
                    jnp.full((_LANES,), j, dtype=jnp.int32), mask=lane0)
                plsc.store_scatter(
                    inv_v,
                    [jnp.full((_LANES,), i16 * _LANES + l, dtype=jnp.int32)],
                    jnp.full((_LANES,), wid * _SORT + slot, dtype=jnp.int32),
                    mask=lane0)
            return c
        lax.fori_loop(0, _B // _LANES, _scat, 0)

        def _gidx(i, c):
            sl = pl.ds(pl.multiple_of(i * _LANES, _LANES), _LANES)
            idxg_v[sl] = xv[sl] + fV
            return c
        lax.fori_loop(0, _B // _LANES, _gidx, 0)

        def _fcg(j, c):
            off = pl.ds(pl.multiple_of(j * _STEP, _STEP), _STEP)
            pltpu.async_copy(fc_hbm.at[idxg_v.at[off]], fcv.at[off], sem_f)
            return c
        lax.fori_loop(0, _B // _STEP, _fcg, 0)

        iota16 = lax.iota(jnp.int32, _LANES)

        def _window(k, seq):
            slot = lax.rem(k, _RING)
            raw_k = base0 + k * _WIN
            fullc = raw_k + _WIN <= _NF
            for s in range(_RING):
                @pl.when(jnp.logical_and(slot == s, fullc))
                def _wait_full(s=s):
                    pltpu.make_async_copy(
                        emb_t.at[:, pl.ds(0, _WIN)], win_v.at[s],
                        sem_s[s]).wait()

                @pl.when(jnp.logical_and(slot == s, jnp.logical_not(fullc)))
                def _wait_tail(s=s):
                    pltpu.make_async_copy(
                        emb_t.at[:, pl.ds(0, 128)],
                        win_v.at[s, :, pl.ds(0, 128)], sem_s[s]).wait()

            start = raw_k
            coff = fV - start
            cnt_k = cnt_smem[k]
            roff = off_smem[k]
            ng32 = lax.shift_right_logical(cnt_k + 31, 5)

            def _group32(g2, seq_in):
                sr = lax.rem(seq_in, _SRING)
                for s in range(_SRING):
                    @pl.when(jnp.logical_and(sr == s, seq_in >= _SRING))
                    def _reuse(s=s):
                        pltpu.make_async_copy(
                            outs_hbm.at[pl.ds(0, 8), :], stage_v.at[s],
                            sem_w[s]).wait()
                for h in range(2):
                    g = g2 * 2 + h
                    sl16 = pl.ds(
                        pl.multiple_of(roff + g * _LANES, _LANES), _LANES)
                    jv = sj_v[sl16]
                    mask = (g * _LANES + iota16) < cnt_k
                    cvec = jv + coff
                    for d in range(_D):
                        dvec = jnp.full((_LANES,), d, dtype=jnp.int32)
                        vals = plsc.load_gather(win_v.at[slot], [dvec, cvec],
                                                mask=mask)
                        wv = (h * _LANES + iota16) * _D + d
                        plsc.store_scatter(
                            stage_v.at[sr],
                            [lax.shift_right_logical(wv, 7),
                             lax.bitwise_and(wv, 127)], vals, mask=mask)
                r0 = lax.shift_right_logical(
                    wid * _SORT + roff + g2 * 32, 2)
                for s in range(_SRING):
                    @pl.when(sr == s)
                    def _go(s=s):
                        pltpu.async_copy(
                            stage_v.at[s],
                            outs_hbm.at[pl.ds(pl.multiple_of(r0, 8), 8), :],
                            sem_w[s])
                return seq_in + 1

            seq = lax.fori_loop(0, ng32, _group32, seq)

            @pl.when(k + _RING < _NWIN)
            def _next():
                _fire(k + _RING)
            return seq

        lax.fori_loop(0, _NWIN, _window, 0)

        for s in range(_SRING):
            pltpu.make_async_copy(outs_hbm.at[pl.ds(0, 8), :],
                                  stage_v.at[s], sem_w[s]).wait()

        pltpu.sync_copy(inv_v, inv_hbm.at[pl.ds(pbase, _B)])
        pltpu.make_async_copy(fc_out.at[pl.ds(pbase, _B)], fcv, sem_f).wait()
        pltpu.sync_copy(fcv, fc_out.at[pl.ds(pbase, _B)])


def _sc_permute_body(outs_hbm, inv_hbm, outp_hbm,
                     invb_v, prix_v, rows_v, pack_v, sem_g, sem_w):
    wid = lax.axis_index("s") * _NC + lax.axis_index("c")
    pbase = pl.multiple_of(wid * (_TOT // _NW), _BCH)
    iota16 = lax.iota(jnp.int32, _LANES)

    def _load(bt):
        slot = lax.rem(bt, 2)
        pltpu.sync_copy(inv_hbm.at[pl.ds(pbase + bt * _BCH, _BCH)],
                        invb_v.at[slot])

        def _shift(i, c):
            sl = pl.ds(pl.multiple_of(i * _LANES, _LANES), _LANES)
            prix_v[slot, sl] = lax.shift_right_logical(invb_v[slot, sl], 2)
            return c
        lax.fori_loop(0, _BCH // _LANES, _shift, 0)
        for q in range(_BCH // _STEP):
            pltpu.async_copy(
                outs_hbm.at[prix_v.at[slot, pl.ds(q * _STEP, _STEP)]],
                rows_v.at[slot, pl.ds(q * _STEP, _STEP), :], sem_g)

    _load(0)

    def _batch(bt, c):
        slot = lax.rem(bt, 2)

        pltpu.make_async_copy(outs_hbm.at[pl.ds(0, _BCH), :],
                              rows_v.at[slot], sem_g).wait()

        @pl.when(bt + 1 < _NBCH)
        def _pre():
            _load(bt + 1)

        def _group(g, c2):
            sl16 = pl.ds(pl.multiple_of(g * _LANES, _LANES), _LANES)
            qv = lax.bitwise_and(invb_v[slot, sl16], 3)
            kv = g * _LANES + iota16
            for d in range(_D):
                offs = kv * 128 + qv * _D + d
                vals = plsc.load_gather(
                    rows_v.at[slot],
                    [lax.shift_right_logical(offs, 7),
                     lax.bitwise_and(offs, 127)])
                wv = kv * _D + d
                plsc.store_scatter(
                    pack_v, [lax.shift_right_logical(wv, 7),
                             lax.bitwise_and(wv, 127)], vals)
            return c2
        lax.fori_loop(0, _BCH // _LANES, _group, 0)

        r0 = lax.shift_right_logical(pbase + bt * _BCH, 2)
        pltpu.sync_copy(pack_v,
                        outp_hbm.at[pl.ds(pl.multiple_of(r0, 64), 64), :])
        return c
    lax.fori_loop(0, _NBCH, _batch, 0)


@functools.lru_cache(maxsize=1)
def _make_sc_kernels():
    mesh = plsc.VectorSubcoreMesh(core_axis_name="c", subcore_axis_name="s")
    cp = pltpu.CompilerParams(use_tc_tiling_on_sc=True,
                              needs_layout_passes=False)
    stream = pl.kernel(
        _sc_stream_body,
        out_type=[
            jax.ShapeDtypeStruct((_SROWS, 128), jnp.float32),
            jax.ShapeDtypeStruct((_TOT,), jnp.int32),
            jax.ShapeDtypeStruct((_TOT,), jnp.float32),
        ],
        mesh=mesh,
        compiler_params=cp,
        scratch_types=[
            pltpu.VMEM((_RING, _D, _WIN), jnp.float32),
            pltpu.VMEM((_B,), jnp.int32),
            pltpu.VMEM((_B,), jnp.int32),
            pltpu.VMEM((_B,), jnp.float32),
            pltpu.VMEM((_SORT,), jnp.int32),
            pltpu.VMEM((_B,), jnp.int32),
            pltpu.VMEM((_SRING, 8, 128), jnp.float32),
            pltpu.SMEM((256,), jnp.int32),
            pltpu.SMEM((256,), jnp.int32),
            pltpu.SMEM((256,), jnp.int32),
        ] + [pltpu.SemaphoreType.DMA] * 9,
    )
    permute = pl.kernel(
        _sc_permute_body,
        out_type=jax.ShapeDtypeStruct((_PROWS, 128), jnp.float32),
        mesh=mesh,
        compiler_params=cp,
        scratch_types=[
            pltpu.VMEM((2, _BCH), jnp.int32),
            pltpu.VMEM((2, _BCH), jnp.int32),
            pltpu.VMEM((2, _BCH, 128), jnp.float32),
            pltpu.VMEM((64, 128), jnp.float32),
            pltpu.SemaphoreType.DMA,
            pltpu.SemaphoreType.DMA,
        ],
    )
    return stream, permute


_BB = 512
_NBB = _B // _BB


def _tc_body(emb_ref, fct_ref, w1_ref, b1_ref, gamma_ref, beta_ref,
             w2_ref, b2_ref, linw_ref, linb_ref, out_ref, h_s, fl_s):
    ii = lax.broadcasted_iota(jnp.int32, (_EO, _D), 0)
    jj = lax.broadcasted_iota(jnp.int32, (_EO, _D), 1)
    S = jnp.where(ii % _D == jj, 1.0, 0.0).astype(jnp.float32)
    w1 = w1_ref[...]
    b1 = b1_ref[...]
    hp = lax.Precision.HIGHEST

    ones_f = jnp.ones((_F, 1), dtype=jnp.float32)
    lin = (lax.dot_general(fct_ref[...], ones_f, (((0,), (0,)), ((), ())),
                           precision=hp) * linw_ref[0, 0] + linb_ref[0, 0])

    def _blk(b, carry):
        rows = pl.ds(b * _BB, _BB)
        e = emb_ref[rows, :]
        ssq = jnp.sum(e * e, axis=1, keepdims=True)
        s = lax.dot(e, S, precision=hp)
        fm = 0.5 * (jnp.sum(s * s, axis=1, keepdims=True) - ssq)
        h_s[rows, :] = lax.dot(e, w1, precision=hp) + b1
        fl_s[rows, :] = fm
        return carry
    lax.fori_loop(0, _NBB, _blk, 0)

    h = h_s[...]
    mean = jnp.mean(h, axis=0, keepdims=True)
    var = jnp.mean(jnp.square(h - mean), axis=0, keepdims=True)
    hn = (h - mean) * lax.rsqrt(var + 1e-5) * gamma_ref[...] + beta_ref[...]
    hr = jnp.maximum(hn, 0.0)
    deep = lax.dot(hr, w2_ref[...], precision=hp) + b2_ref[0, 0]

    out_ref[...] = fl_s[...] + lin + deep


def _tc_forward(emb_flat, fct, W1, b1, gamma, beta, W2, b2, lin_W, lin_b):
    return pl.pallas_call(
        _tc_body,
        out_shape=jax.ShapeDtypeStruct((_B, 1), jnp.float32),
        scratch_shapes=[
            pltpu.VMEM((_B, _H), jnp.float32),
            pltpu.VMEM((_B, 1), jnp.float32),
        ],
    )(emb_flat, fct,
      W1, b1.reshape(1, _H), gamma.reshape(1, _H), beta.reshape(1, _H),
      W2, b2.reshape(1, 1), lin_W, lin_b.reshape(1, 1))


def kernel(emb_table, fc_table, lin_W, lin_b, W1, b1, gamma, beta, W2, b2, x):
    emb_t = emb_table.T
    xt_flat = x.T.reshape(-1)
    fc_flat = fc_table.reshape(-1)
    stream, permute = _make_sc_kernels()
    out_s, inv_fm, fc_vals = stream(emb_t, fc_flat, xt_flat)
    inv_sm = inv_fm.reshape(_F, _B).T.reshape(-1)
    out_p = permute(out_s, inv_sm)
    emb_flat = out_p.reshape(_B, _EO)
    fct = fc_vals.reshape(_F, _B)
    out = _tc_forward(emb_flat, fct, W1, b1, gamma, beta, W2, b2, lin_W, lin_b)
    return out.reshape(-1)
